# baseline (speedup 1.0000x reference)
import functools
import math

import jax
import jax.numpy as jnp
from jax import lax
from jax.experimental import pallas as pl
from jax.experimental.pallas import tpu as pltpu
from jax.experimental.pallas import tpu_sc as plsc

D_MODEL = 64
SCALE = math.sqrt(D_MODEL)
L = 16
NT = 16
VT = 2048
TAIL = 128
SB = 256


@functools.lru_cache(maxsize=None)
def _make(vocab, d, s_len, b_len):
    n_pos = s_len * b_len
    n_e = n_pos // NT
    n_chunk = (vocab + NT * VT - 1) // (NT * VT)
    n_rk = 512
    nch_core = d // 2
    n_g = nch_core // 8
    aligned_v = (vocab // 128) * 128
    vtail = vocab - aligned_v
    offcap = aligned_v - VT + (128 if vtail else 0)
    half_s = s_len // 2
    plane_w = half_s * 8192
    dump = plane_w
    inbox_cap = n_pos + SB

    mesh = plsc.VectorSubcoreMesh(core_axis_name="c", subcore_axis_name="s")

    @functools.partial(
        pl.kernel,
        mesh=mesh,
        out_type=jax.ShapeDtypeStruct((b_len * s_len * d,), jnp.float32),
        compiler_params=pltpu.CompilerParams(
            use_tc_tiling_on_sc=True, needs_layout_passes=False,
            disable_bounds_checks=True),
        scratch_types=[
            pltpu.VMEM((n_e,), jnp.int32),
            pltpu.VMEM((8, VT), jnp.float32),
            pltpu.VMEM((n_rk * L,), jnp.int32),
            pltpu.VMEM((NT * n_rk,), jnp.int32),
            pltpu.VMEM((n_rk + 1,), jnp.int32),
            pltpu.VMEM((n_rk,), jnp.int32),
            pltpu.VMEM((n_e // 2,), jnp.int32),
            pltpu.VMEM((SB,), jnp.int32),
            [pltpu.VMEM((SB,), jnp.float32) for _ in range(8)],
            [pltpu.VMEM((SB,), jnp.int32) for _ in range(8)],
            pltpu.VMEM((4096,), jnp.float32),
            pltpu.VMEM_SHARED((inbox_cap,), jnp.int32),
            pltpu.VMEM_SHARED((NT * n_rk,), jnp.int32),
            pltpu.VMEM_SHARED((plane_w + 1040,), jnp.float32),
            pltpu.SemaphoreType.DMA,
        ],
    )
    def k(tt, xf, out, raw, chunk, hist, ttot, rkbase, pbase, pubi,
          pkwin, vwin, owin, wbuf, inbox, board, plane, sem):
        core = lax.axis_index("c")
        tile = lax.axis_index("s")
        j0 = tile * n_e

        pltpu.sync_copy(xf.at[pl.ds(j0, n_e)], raw)

        def zero_hist(v, c):
            hist[pl.ds(v * L, L)] = jnp.zeros((L,), jnp.int32)
            return c

        lax.fori_loop(0, n_rk, zero_hist, 0)

        def hist_pass(v, c):
            iota = lax.iota(jnp.int32, L)
            xv = raw[pl.ds(v * L, L)]
            blk = lax.shift_right_logical(xv, 11)
            rk = lax.shift_left(blk & 15, 5) | lax.shift_right_logical(blk, 4)
            plsc.addupdate_scatter(
                hist, [lax.shift_left(rk, 4) + iota],
                jnp.full((L,), 1, jnp.int32))
            return c

        lax.fori_loop(0, n_e // L, hist_pass, 0)

        def tot_pass(v, c):
            iota = lax.iota(jnp.int32, L)
            hv = hist[pl.ds(v * L, L)]
            t = jnp.max(plsc.cumsum(hv))
            plsc.store_scatter(
                ttot, [jnp.full((L,), v, jnp.int32)],
                jnp.full((L,), 1, jnp.int32) * t,
                mask=iota == 0)
            return c

        lax.fori_loop(0, n_rk, tot_pass, 0)
        pltpu.sync_copy(ttot.at[pl.ds(0, n_rk)],
                        board.at[pl.ds(tile * n_rk, n_rk)])
        plsc.subcore_barrier()
        pltpu.sync_copy(board, ttot)

        def base_pass(v, run):
            acc = jnp.zeros((L,), jnp.int32)
            for p in range(NT):
                acc = acc + ttot[pl.ds(p * n_rk + v * L, L)]
            cs = plsc.cumsum(acc)
            excl = cs - acc + run
            iota = lax.iota(jnp.int32, L)
            plsc.store_scatter(rkbase, [iota + v * L], excl)
            return run + jnp.max(cs)

        total_fin = lax.fori_loop(0, n_rk // L, base_pass, 0)
        plsc.store_scatter(
            rkbase, [jnp.full((L,), n_rk, jnp.int32)],
            jnp.full((L,), 1, jnp.int32) * total_fin,
            mask=lax.iota(jnp.int32, L) == 0)

        def pbase_pass(v, c):
            acc = rkbase[pl.ds(v * L, L)]
            for p in range(NT):
                tp = ttot[pl.ds(p * n_rk + v * L, L)]
                acc = acc + jnp.where(
                    jnp.full((L,), p, jnp.int32) < tile, tp, 0)
            pbase[pl.ds(v * L, L)] = acc
            return c

        lax.fori_loop(0, n_rk // L, pbase_pass, 0)

        def cur_pass(v, c):
            hv = hist[pl.ds(v * L, L)]
            cs = plsc.cumsum(hv)
            lane_excl = cs - hv
            pb = plsc.load_gather(pbase, [jnp.full((L,), v, jnp.int32)])
            hist[pl.ds(v * L, L)] = lane_excl + pb
            return c

        lax.fori_loop(0, n_rk, cur_pass, 0)

        for h in range(2):
            h0 = h * (n_e // 2)

            def pub_pass(v, c, h0=h0):
                iota = lax.iota(jnp.int32, L)
                xv = raw[pl.ds(h0 + v * L, L)]
                blk = lax.shift_right_logical(xv, 11)
                rk = (lax.shift_left(blk & 15, 5)
                      | lax.shift_right_logical(blk, 4))
                gidx = lax.shift_left(rk, 4) + iota
                base = plsc.load_gather(hist, [gidx])
                jv = j0 + h0 + v * L + iota
                pk = lax.shift_left(jv, 11) | (xv & (VT - 1))
                raw[pl.ds(h0 + v * L, L)] = pk
                pubi[pl.ds(v * L, L)] = base
                plsc.addupdate_scatter(
                    hist, [gidx], jnp.full((L,), 1, jnp.int32))
                return c

            lax.fori_loop(0, n_e // 2 // L, pub_pass, 0)
            pltpu.async_copy(
                raw.at[pl.ds(h0, n_e // 2)], inbox.at[pubi], sem).wait()
        plsc.subcore_barrier()

        for g in range(n_g):
          for sh in range(2):
            ch0 = core * nch_core + 8 * g
            slo = sh * half_s

            def q_body(q, c0, ch0=ch0, slo=slo):
                raw_off = q * (NT * VT) + tile * VT
                off_t = jnp.minimum(raw_off, offcap)
                delta = raw_off - off_t
                pltpu.sync_copy(
                    tt.at[pl.ds(ch0, 8), pl.ds(off_t, VT)], chunk)

                rk = lax.shift_left(tile, 5) | q
                lov = plsc.load_gather(
                    rkbase, [jnp.full((L,), rk, jnp.int32)])
                hiv = plsc.load_gather(
                    rkbase, [jnp.full((L,), rk + 1, jnp.int32)])
                lo = lov[0]
                hi = hiv[0]
                lo8 = lax.shift_left(lax.shift_right_logical(lo, 3), 3)
                n_w = (hi - lo8 + SB - 1) // SB

                def win_body(w, c1, lo=lo, lo8=lo8, hi=hi, delta=delta,
                             slo=slo):
                    ws = pl.multiple_of(lo8 + w * SB, 8)
                    pltpu.sync_copy(inbox.at[pl.ds(ws, SB)], pkwin)

                    def vec_body(i, c2, ws=ws, lo=lo, hi=hi, delta=delta,
                                 slo=slo):
                        iota = lax.iota(jnp.int32, L)
                        pk = pkwin[pl.ds(i * L, L)]
                        jv = lax.shift_right_logical(pk, 11)
                        lv = jnp.minimum((pk & (VT - 1)) + delta, VT - 1)
                        sv = lax.shift_right_logical(jv, 10) - slo
                        bv = jv & (b_len - 1)
                        base = (lax.shift_left(sv, 13)
                                + lax.shift_left(
                                    lax.shift_right_logical(bv, 7), 10)
                                + (bv & 127))
                        pos = ws + i * L + iota
                        msk = ((pos >= lo) & (pos < hi)
                               & (sv >= 0) & (sv < half_s))
                        sbase = jnp.where(msk, base,
                                          jnp.full((L,), dump, jnp.int32))
                        for c in range(8):
                            cv = jnp.full((L,), c, jnp.int32)
                            val = plsc.load_gather(chunk, [cv, lv])
                            vwin[c][pl.ds(i * L, L)] = val * SCALE
                            owin[c][pl.ds(i * L, L)] = sbase + c * 128
                        return c2

                    lax.fori_loop(0, SB // L, vec_body, 0)
                    cps = [pltpu.async_copy(vwin[c], plane.at[owin[c]], sem)
                           for c in range(8)]
                    for cp in cps:
                        cp.wait()
                    return c1

                lax.fori_loop(0, n_w, win_body, 0)
                return c0

            lax.fori_loop(0, n_chunk, q_body, 0)
            plsc.subcore_barrier()

            def wr_body(i, c0, g=g, slo=slo):
                s = tile + NT * i

                @pl.when(s < half_s)
                def _(s=s, g=g, slo=slo):
                    for hh in range(2):
                        pltpu.sync_copy(
                            plane.at[pl.ds(s * 8192 + hh * 4096, 4096)],
                            wbuf)
                        pltpu.sync_copy(
                            wbuf,
                            out.at[pl.ds((s + slo) * (d * b_len)
                                         + (core * n_g + g) * 8192
                                         + hh * 4096, 4096)])

                return c0

            lax.fori_loop(0, (half_s + NT - 1) // NT, wr_body, 0)
            plsc.subcore_barrier()

    return k


def kernel(x, table):
    batch, seq = x.shape
    vocab, d = table.shape
    tt = table.T
    xf = x.T.reshape(-1)
    flat = _make(vocab, d, seq, batch)(tt, xf)
    r = flat.reshape(seq, 8, 8, 8, 128)
    return r.transpose(2, 4, 0, 1, 3).reshape(batch, seq, d)


# speedup vs baseline: 3.9311x; 3.9311x over previous
import functools
import math

import jax
import jax.numpy as jnp
from jax import lax
from jax.experimental import pallas as pl
from jax.experimental.pallas import tpu as pltpu
from jax.experimental.pallas import tpu_sc as plsc

D_MODEL = 64
_SCALE = math.sqrt(D_MODEL)


@functools.lru_cache(maxsize=None)
def _make_gather(vocab: int, d: int, b: int):
    info = plsc.get_sparse_core_info()
    nc, ns, lanes = info.num_cores, info.num_subcores, info.num_lanes
    nw = nc * ns
    assert b % nw == 0
    b_per_w = b // nw
    ch = 800
    while b_per_w % ch:
        ch //= 2
    n_ch = b_per_w // ch

    mesh = plsc.VectorSubcoreMesh(core_axis_name="c", subcore_axis_name="s")

    @functools.partial(
        pl.kernel,
        mesh=mesh,
        out_type=jax.ShapeDtypeStruct((b, d), jnp.float32),
        compiler_params=pltpu.CompilerParams(use_tc_tiling_on_sc=False),
        scratch_types=[
            pltpu.VMEM((b_per_w,), jnp.int32),
            pltpu.VMEM((ch, d), jnp.float32),
            pltpu.SemaphoreType.DMA,
        ],
    )
    def gather_kernel(idx_hbm, table_hbm, out_hbm, idx_v, rows_v, sem):
        wid = lax.axis_index("s") * nc + lax.axis_index("c")
        base = wid * b_per_w
        pltpu.sync_copy(idx_hbm.at[pl.ds(base, b_per_w)], idx_v)

        def chunk_body(c, carry):
            off = pl.multiple_of(c * ch, 8)
            pltpu.async_copy(
                table_hbm.at[idx_v.at[pl.ds(off, ch)]], rows_v, sem
            ).wait()

            def scale_body(r, carry2):
                for j in range(d // lanes):
                    sl = pl.ds(j * lanes, lanes)
                    rows_v[r, sl] = rows_v[r, sl] * _SCALE
                return carry2

            lax.fori_loop(0, ch, scale_body, 0)
            pltpu.sync_copy(rows_v, out_hbm.at[pl.ds(base + off, ch)])
            return carry

        lax.fori_loop(0, n_ch, chunk_body, 0)

    return gather_kernel


def kernel(x, table):
    batch, seq = x.shape
    vocab, d = table.shape
    b = batch * seq
    xf = x.reshape(b).astype(jnp.int32)
    out = _make_gather(vocab, d, b)(xf, table)
    return out.reshape(batch, seq, d)
